# baseline (speedup 1.0000x reference)
import functools

import jax
import jax.numpy as jnp
from jax import lax
from jax.experimental import pallas as pl
from jax.experimental.pallas import tpu as pltpu
from jax.experimental.pallas import tpu_sc as plsc

NC = 2
NS = 16
L = 16
NW = NC * NS

CH = 128
BLKH = 1000


def _pack_halves(x):
    n, k = x.shape
    h = n // 2
    assert h % BLKH == 0

    def body(a_ref, b_ref, o_ref):
        o_ref[...] = jnp.concatenate([a_ref[...], b_ref[...]], axis=1)

    nb = h // BLKH
    return pl.pallas_call(
        body,
        grid=(nb,),
        in_specs=[
            pl.BlockSpec((BLKH, k), lambda i: (i, 0)),
            pl.BlockSpec((BLKH, k), lambda i, nb=nb: (i + nb, 0)),
        ],
        out_specs=pl.BlockSpec((BLKH, 2 * k), lambda i: (i, 0)),
        out_shape=jax.ShapeDtypeStruct((h, 2 * k), jnp.float32),
    )(x, x)


def _make_sc_kernel(B, K, HU, HM):
    assert B % NW == 0
    bw = B // NW
    nch = bw // CH
    assert nch * CH == bw and K % L == 0
    K2 = 2 * K

    mesh = plsc.VectorSubcoreMesh(core_axis_name="c", subcore_axis_name="s")

    @functools.partial(
        pl.kernel,
        mesh=mesh,
        out_type=jax.ShapeDtypeStruct((B,), jnp.float32),
        compiler_params=pltpu.CompilerParams(
            needs_layout_passes=False, use_tc_tiling_on_sc=True),
        scratch_types=[
            pltpu.VMEM((bw,), jnp.int32),
            pltpu.VMEM((bw,), jnp.int32),
            pltpu.VMEM((bw,), jnp.int32),
            pltpu.VMEM((bw,), jnp.int32),
            pltpu.VMEM((CH, K2), jnp.float32),
            pltpu.VMEM((CH, K2), jnp.float32),
            pltpu.VMEM((bw,), jnp.float32),
            pltpu.SemaphoreType.DMA,
        ],
    )
    def body(users_hbm, movies_hbm, upairs_hbm, mpairs_hbm, out_hbm,
             ug, mg, ub, mb, ubuf, mbuf, outv, sem):
        wid = lax.axis_index("s") * NC + lax.axis_index("c")
        base = wid * bw
        pltpu.sync_copy(users_hbm.at[pl.ds(base, bw)], ug)
        pltpu.sync_copy(movies_hbm.at[pl.ds(base, bw)], mg)
        lane = lax.iota(jnp.int32, L)

        def split(j, _):
            sl = pl.ds(j * L, L)
            uv, mv = ug[sl], mg[sl]
            ub[sl] = jnp.where(uv < HU, 0, K)
            mb[sl] = jnp.where(mv < HM, 0, K)
            ug[sl] = jnp.where(uv < HU, uv, uv - HU)
            mg[sl] = jnp.where(mv < HM, mv, mv - HM)
            return _

        lax.fori_loop(0, bw // L, split, 0)

        for c in range(nch):
            pltpu.async_copy(
                upairs_hbm.at[ug.at[pl.ds(c * CH, CH)]], ubuf, sem).wait()
            pltpu.async_copy(
                mpairs_hbm.at[mg.at[pl.ds(c * CH, CH)]], mbuf, sem).wait()

            def group(g, _, c=c):
                sl = pl.ds(c * CH + g * L, L)
                ubv, mbv = ub[sl], mb[sl]
                accv = jnp.zeros((L,), jnp.float32)
                for i in range(L):
                    r = g * L + i
                    ubase, mbase = ubv[i], mbv[i]
                    p = (ubuf[r, pl.ds(ubase, L)] * mbuf[r, pl.ds(mbase, L)])
                    for k in range(L, K, L):
                        p += (ubuf[r, pl.ds(ubase + k, L)]
                              * mbuf[r, pl.ds(mbase + k, L)])
                    accv = jnp.where(lane == i, plsc.cumsum(p)[L - 1], accv)
                outv[pl.ds(c * CH + g * L, L)] = accv
                return _

            lax.fori_loop(0, CH // L, group, 0)

        pltpu.sync_copy(outv, out_hbm.at[pl.ds(base, bw)])

    return body


def kernel(users, movies, user_emb, movie_emb):
    B = users.shape[0]
    K = user_emb.shape[1]
    upairs = _pack_halves(user_emb)
    mpairs = _pack_halves(movie_emb)
    return _make_sc_kernel(B, K, user_emb.shape[0] // 2,
                           movie_emb.shape[0] // 2)(
        users.astype(jnp.int32), movies.astype(jnp.int32), upairs, mpairs)


# speedup vs baseline: 1.3387x; 1.3387x over previous
import functools

import jax
import jax.numpy as jnp
from jax import lax
from jax.experimental import pallas as pl
from jax.experimental.pallas import tpu as pltpu
from jax.experimental.pallas import tpu_sc as plsc

NC = 2
NS = 16
L = 16
NW = NC * NS

CH = 128


def _make_sc_kernel(B, K):
    assert B % NW == 0
    bw = B // NW
    nch = bw // CH
    assert nch * CH == bw and K % L == 0

    mesh = plsc.VectorSubcoreMesh(core_axis_name="c", subcore_axis_name="s")

    @functools.partial(
        pl.kernel,
        mesh=mesh,
        out_type=jax.ShapeDtypeStruct((B,), jnp.float32),
        compiler_params=pltpu.CompilerParams(
            needs_layout_passes=False, use_tc_tiling_on_sc=False),
        scratch_types=[
            pltpu.VMEM((nch, CH), jnp.int32),
            pltpu.VMEM((nch, CH), jnp.int32),
            pltpu.VMEM((CH, K), jnp.float32),
            pltpu.VMEM((CH, K), jnp.float32),
            pltpu.VMEM((bw,), jnp.float32),
            pltpu.SemaphoreType.DMA,
        ],
    )
    def body(users_hbm, movies_hbm, uemb_hbm, memb_hbm, out_hbm,
             uidx, midx, urows, mrows, outv, sem):
        wid = lax.axis_index("s") * NC + lax.axis_index("c")
        pltpu.sync_copy(users_hbm.at[wid], uidx)
        pltpu.sync_copy(movies_hbm.at[wid], midx)
        lane = lax.iota(jnp.int32, 16)

        for c in range(nch):
            pltpu.async_copy(uemb_hbm.at[uidx.at[c]], urows, sem).wait()
            pltpu.async_copy(memb_hbm.at[midx.at[c]], mrows, sem).wait()

            def group(g, _, c=c):
                accv = jnp.zeros((L,), jnp.float32)
                for i in range(L):
                    r = g * L + i
                    p = urows[r, pl.ds(0, L)] * mrows[r, pl.ds(0, L)]
                    for k in range(L, K, L):
                        p += urows[r, pl.ds(k, L)] * mrows[r, pl.ds(k, L)]
                    accv = jnp.where(lane == i, plsc.cumsum(p)[15], accv)
                outv[pl.ds(c * CH + g * L, L)] = accv
                return _

            lax.fori_loop(0, CH // L, group, 0)

        pltpu.sync_copy(outv, out_hbm.at[pl.ds(wid * bw, bw)])

    return body


def kernel(users, movies, user_emb, movie_emb):
    B = users.shape[0]
    K = user_emb.shape[1]
    bw = B // NW
    nch = bw // CH
    u3 = users.astype(jnp.int32).reshape(NW, nch, CH)
    m3 = movies.astype(jnp.int32).reshape(NW, nch, CH)
    ulin = lax.optimization_barrier(user_emb.reshape(-1))
    mlin = lax.optimization_barrier(movie_emb.reshape(-1))
    return _make_sc_kernel(B, K)(
        u3, m3, ulin.reshape(user_emb.shape), mlin.reshape(movie_emb.shape))
